# baseline (speedup 1.0000x reference)
import functools

import jax
import jax.numpy as jnp
from jax import lax
from jax.experimental import pallas as pl
from jax.experimental.pallas import tpu as pltpu
from jax.experimental.pallas import tpu_sc as plsc

N = 100000
D = 128
B = 16384
IN_DIM = 192
NC, NS = 2, 16
NW = NC * NS
BPW = B // NW
NA = 25
R = N // NA
CH = 128
NLIST = 4096
WR = 160
WN = R // WR

_mesh = plsc.VectorSubcoreMesh(core_axis_name="c", subcore_axis_name="s")


@functools.partial(
    pl.kernel,
    out_type=jax.ShapeDtypeStruct((B, D), jnp.float32),
    mesh=_mesh,
    scratch_types=[
        pltpu.VMEM((BPW,), jnp.int32),
        pltpu.VMEM((BPW, D), jnp.float32),
        pltpu.SemaphoreType.DMA,
    ],
)
def _sc_gather(mem_hbm, ids_hbm, h_hbm, idx_v, rows_v, sem):
    wid = lax.axis_index("s") * NC + lax.axis_index("c")
    base = wid * BPW
    pltpu.sync_copy(ids_hbm.at[pl.ds(base, BPW)], idx_v)
    pltpu.async_copy(mem_hbm.at[idx_v], rows_v, sem).wait()
    pltpu.sync_copy(rows_v, h_hbm.at[pl.ds(base, BPW)])


@functools.partial(
    pl.kernel,
    out_type=(jax.ShapeDtypeStruct((NA * NLIST,), jnp.int32),
              jax.ShapeDtypeStruct((NA * NLIST,), jnp.int32),
              jax.ShapeDtypeStruct((NA * 16,), jnp.int32),
              jax.ShapeDtypeStruct((N,), jnp.float32)),
    mesh=_mesh,
    scratch_types=[
        pltpu.VMEM((B,), jnp.int32),
        pltpu.VMEM((B,), jnp.float32),
        pltpu.VMEM((R,), jnp.int32),
        pltpu.VMEM((NLIST,), jnp.int32),
        pltpu.VMEM((NLIST,), jnp.int32),
        pltpu.VMEM((R,), jnp.float32),
        pltpu.VMEM((16,), jnp.int32),
        pltpu.SemaphoreType.DMA,
        pltpu.SemaphoreType.DMA,
        pltpu.SemaphoreType.DMA,
    ],
    compiler_params=pltpu.CompilerParams(needs_layout_passes=False),
)
def _sc_plan(ids, ts, lu, out_nodes, out_wins, out_counts, out_lu,
             ids_v, ts_v, win_v, nodes_l, win_l, luv, cnt_v,
             sem_ids, sem_ts, sem_lu):
    wid = lax.axis_index("s") * NC + lax.axis_index("c")

    @pl.when(wid < NA)
    def _active():
        nbase = wid * R
        iota16 = lax.iota(jnp.int32, 16)

        cp_ids = pltpu.async_copy(ids, ids_v, sem_ids)
        cp_ts = pltpu.async_copy(ts, ts_v, sem_ts)
        cp_lu = pltpu.async_copy(lu.at[pl.ds(nbase, R)], luv, sem_lu)
        cp_ids.wait()

        neg1 = jnp.full((16,), -1, jnp.int32)

        def _init(i, _):
            win_v[pl.ds(i * 16, 16)] = neg1
            return 0
        lax.fori_loop(0, R // 16, _init, 0)

        def _scan(j, _):
            ids16 = ids_v[pl.ds(j * 16, 16)]
            batch = j * 16 + iota16
            _, islast = plsc.scan_count(ids16)
            loc = ids16 - nbase
            m1 = islast & (loc >= 0) & (loc < R)
            plsc.store_scatter(win_v, [jnp.clip(loc, 0, R - 1)], batch,
                               mask=m1)
            return 0
        lax.fori_loop(0, B // 16, _scan, 0)

        def _comp(i, cnt):
            w16 = win_v[pl.ds(i * 16, 16)]
            m = w16 >= 0
            n16 = nbase + i * 16 + iota16
            plsc.store_compressed(nodes_l.at[pl.ds(cnt, 16)], n16, mask=m)
            plsc.store_compressed(win_l.at[pl.ds(cnt, 16)], w16, mask=m)
            return cnt + jnp.max(plsc.all_reduce_population_count(m))
        cnt = lax.fori_loop(0, R // 16, _comp, jnp.int32(0))

        cp_ts.wait()
        cp_lu.wait()

        def _blend(i, _):
            w16 = win_v[pl.ds(i * 16, 16)]
            m = w16 >= 0
            tsv = plsc.load_gather(ts_v, [jnp.maximum(w16, 0)])
            cur = luv[pl.ds(i * 16, 16)]
            luv[pl.ds(i * 16, 16)] = jnp.where(m, tsv, cur)
            return 0
        lax.fori_loop(0, R // 16, _blend, 0)
        pltpu.sync_copy(luv, out_lu.at[pl.ds(nbase, R)])

        zero16 = jnp.zeros((16,), jnp.int32)
        padn = nodes_l[pl.ds(0, 16)].at[zero16].get(mode="promise_in_bounds")
        padw = win_l[pl.ds(0, 16)].at[zero16].get(mode="promise_in_bounds")
        npad = ((cnt + CH - 1) // CH) * CH

        def _pad(j, _):
            keep = (j * 16 + iota16) < cnt
            nodes_l[pl.ds(j * 16, 16)] = jnp.where(
                keep, nodes_l[pl.ds(j * 16, 16)], padn)
            win_l[pl.ds(j * 16, 16)] = jnp.where(
                keep, win_l[pl.ds(j * 16, 16)], padw)
            return 0
        lax.fori_loop(cnt // 16, npad // 16, _pad, 0)

        cnt_v[pl.ds(0, 16)] = jnp.full((16,), 1, jnp.int32) * cnt
        pltpu.sync_copy(cnt_v, out_counts.at[pl.ds(wid * 16, 16)])
        pltpu.sync_copy(nodes_l, out_nodes.at[pl.ds(wid * NLIST, NLIST)])
        pltpu.sync_copy(win_l, out_wins.at[pl.ds(wid * NLIST, NLIST)])


@functools.partial(
    pl.kernel,
    out_type=jax.ShapeDtypeStruct((N, D), jnp.float32),
    mesh=_mesh,
    scratch_types=[
        pltpu.VMEM((NLIST,), jnp.int32),
        pltpu.VMEM((NLIST,), jnp.int32),
        pltpu.VMEM((16,), jnp.int32),
        pltpu.VMEM((CH,), jnp.int32),
        pltpu.VMEM((CH,), jnp.int32),
        pltpu.VMEM((CH, D), jnp.float32),
        pltpu.VMEM((WR, D), jnp.float32),
        pltpu.VMEM((WR, D), jnp.float32),
        pltpu.SemaphoreType.DMA,
        pltpu.SemaphoreType.DMA,
        pltpu.SemaphoreType.DMA,
        pltpu.SemaphoreType.DMA,
        pltpu.SemaphoreType.DMA,
        pltpu.SemaphoreType.DMA,
        pltpu.SemaphoreType.DMA,
        pltpu.SemaphoreType.DMA,
        pltpu.SemaphoreType.DMA,
    ],
    compiler_params=pltpu.CompilerParams(needs_layout_passes=False),
)
def _sc_commit(mem, upd, nodes, wins, counts, out_mem,
               nodes_l, win_l, cnt_v, wchunk, nchunk, rowbuf,
               cbuf_a, cbuf_b,
               sem_n, sem_w, sem_c, sem_g, sem_s,
               sem_ga, sem_gb, sem_sa, sem_sb):
    wid = lax.axis_index("s") * NC + lax.axis_index("c")

    @pl.when(wid < NA)
    def _active():
        nbase = wid * R

        cp_n = pltpu.async_copy(nodes.at[pl.ds(wid * NLIST, NLIST)],
                                nodes_l, sem_n)
        cp_w = pltpu.async_copy(wins.at[pl.ds(wid * NLIST, NLIST)],
                                win_l, sem_w)
        cp_c = pltpu.async_copy(counts.at[pl.ds(wid * 16, 16)], cnt_v,
                                sem_c)

        bufs = (cbuf_a, cbuf_b)
        gsems = (sem_ga, sem_gb)
        ssems = (sem_sa, sem_sb)
        gd = [None] * WN
        sd = [None] * WN
        for k in range(WN):
            b = k & 1
            if k >= 2:
                sd[k - 2].wait()
            gd[k] = pltpu.async_copy(
                mem.at[pl.ds(nbase + k * WR, WR)], bufs[b], gsems[b])
            if k >= 1:
                gd[k - 1].wait()
                sd[k - 1] = pltpu.async_copy(
                    bufs[(k - 1) & 1],
                    out_mem.at[pl.ds(nbase + (k - 1) * WR, WR)],
                    ssems[(k - 1) & 1])
        gd[WN - 1].wait()
        sd[WN - 1] = pltpu.async_copy(
            bufs[(WN - 1) & 1],
            out_mem.at[pl.ds(nbase + (WN - 1) * WR, WR)],
            ssems[(WN - 1) & 1])
        sd[WN - 2].wait()
        sd[WN - 1].wait()

        cp_n.wait()
        cp_w.wait()
        cp_c.wait()
        cnt = jnp.max(cnt_v[pl.ds(0, 16)])
        npad = ((cnt + CH - 1) // CH) * CH

        def _chunk(k, _):
            off = k * CH
            for t in range(CH // 16):
                wchunk[pl.ds(t * 16, 16)] = win_l[pl.ds(off + t * 16, 16)]
                nchunk[pl.ds(t * 16, 16)] = nodes_l[pl.ds(off + t * 16, 16)]
            pltpu.async_copy(upd.at[wchunk], rowbuf, sem_g).wait()
            pltpu.async_copy(rowbuf, out_mem.at[nchunk], sem_s).wait()
            return 0

        @pl.when(cnt > 0)
        def _scatter_phase():
            lax.fori_loop(0, npad // CH, _chunk, 0)


def _gru_body(x_ref, h_ref, wih_ref, whh_ref, bih_ref, bhh_ref, out_ref):
    x = x_ref[...]
    h = h_ref[...]
    gi = jnp.dot(x, wih_ref[...], preferred_element_type=jnp.float32)
    gh = jnp.dot(h, whh_ref[...], preferred_element_type=jnp.float32)
    gi = gi + bih_ref[...]
    gh = gh + bhh_ref[...]
    r = jax.nn.sigmoid(gi[:, :D] + gh[:, :D])
    z = jax.nn.sigmoid(gi[:, D:2 * D] + gh[:, D:2 * D])
    n = jnp.tanh(gi[:, 2 * D:] + r * gh[:, 2 * D:])
    out_ref[...] = (1.0 - z) * n + z * h


_RB = 2048


def _gru(messages, h, W_ihT, W_hhT, b_ih2, b_hh2):
    return pl.pallas_call(
        _gru_body,
        grid=(B // _RB,),
        in_specs=[
            pl.BlockSpec((_RB, IN_DIM), lambda i: (i, 0)),
            pl.BlockSpec((_RB, D), lambda i: (i, 0)),
            pl.BlockSpec((IN_DIM, 3 * D), lambda i: (0, 0)),
            pl.BlockSpec((D, 3 * D), lambda i: (0, 0)),
            pl.BlockSpec((1, 3 * D), lambda i: (0, 0)),
            pl.BlockSpec((1, 3 * D), lambda i: (0, 0)),
        ],
        out_specs=pl.BlockSpec((_RB, D), lambda i: (i, 0)),
        out_shape=jax.ShapeDtypeStruct((B, D), jnp.float32),
    )(messages, h, W_ihT, W_hhT, b_ih2, b_hh2)


def kernel(node_ids, messages, timestamps, memory, last_update, W_ih, W_hh, b_ih, b_hh):
    ids = node_ids.astype(jnp.int32)
    h = _sc_gather(memory, ids)
    plan_nodes, plan_wins, plan_counts, new_last_update = _sc_plan(
        ids, timestamps, last_update)
    updated = _gru(messages, h, W_ih.T, W_hh.T,
                   b_ih.reshape(1, -1), b_hh.reshape(1, -1))
    new_memory = _sc_commit(memory, updated, plan_nodes, plan_wins,
                            plan_counts)
    return new_memory, new_last_update


# speedup vs baseline: 1.0928x; 1.0928x over previous
import functools

import jax
import jax.numpy as jnp
from jax import lax
from jax.experimental import pallas as pl
from jax.experimental.pallas import tpu as pltpu
from jax.experimental.pallas import tpu_sc as plsc

N = 100000
D = 128
B = 16384
IN_DIM = 192
NC, NS = 2, 16
NW = NC * NS
BPW = B // NW
NA = 25
R = N // NA
CH = 128
NLIST = 4096
WR = 160
WN = R // WR

_mesh = plsc.VectorSubcoreMesh(core_axis_name="c", subcore_axis_name="s")


@functools.partial(
    pl.kernel,
    out_type=jax.ShapeDtypeStruct((B, D), jnp.float32),
    mesh=_mesh,
    scratch_types=[
        pltpu.VMEM((BPW,), jnp.int32),
        pltpu.VMEM((BPW, D), jnp.float32),
        pltpu.SemaphoreType.DMA,
    ],
)
def _sc_gather(mem_hbm, ids_hbm, h_hbm, idx_v, rows_v, sem):
    wid = lax.axis_index("s") * NC + lax.axis_index("c")
    base = wid * BPW
    pltpu.sync_copy(ids_hbm.at[pl.ds(base, BPW)], idx_v)
    pltpu.async_copy(mem_hbm.at[idx_v], rows_v, sem).wait()
    pltpu.sync_copy(rows_v, h_hbm.at[pl.ds(base, BPW)])


@functools.partial(
    pl.kernel,
    out_type=(jax.ShapeDtypeStruct((NA * NLIST,), jnp.int32),
              jax.ShapeDtypeStruct((NA * NLIST,), jnp.int32),
              jax.ShapeDtypeStruct((NA * 16,), jnp.int32),
              jax.ShapeDtypeStruct((N,), jnp.float32)),
    mesh=_mesh,
    scratch_types=[
        pltpu.VMEM((B,), jnp.int32),
        pltpu.VMEM((B,), jnp.float32),
        pltpu.VMEM((R,), jnp.int32),
        pltpu.VMEM((NLIST,), jnp.int32),
        pltpu.VMEM((NLIST,), jnp.int32),
        pltpu.VMEM((R,), jnp.float32),
        pltpu.VMEM((16,), jnp.int32),
        pltpu.SemaphoreType.DMA,
        pltpu.SemaphoreType.DMA,
        pltpu.SemaphoreType.DMA,
    ],
    compiler_params=pltpu.CompilerParams(needs_layout_passes=False),
)
def _sc_plan(ids, ts, lu, out_nodes, out_wins, out_counts, out_lu,
             ids_v, ts_v, win_v, nodes_l, win_l, luv, cnt_v,
             sem_ids, sem_ts, sem_lu):
    wid = lax.axis_index("s") * NC + lax.axis_index("c")

    @pl.when(wid < NA)
    def _active():
        nbase = wid * R
        iota16 = lax.iota(jnp.int32, 16)

        cp_ids = pltpu.async_copy(ids, ids_v, sem_ids)
        cp_ts = pltpu.async_copy(ts, ts_v, sem_ts)
        cp_lu = pltpu.async_copy(lu.at[pl.ds(nbase, R)], luv, sem_lu)
        cp_ids.wait()

        neg1 = jnp.full((16,), -1, jnp.int32)

        def _init(i, _):
            win_v[pl.ds(i * 16, 16)] = neg1
            return 0
        lax.fori_loop(0, R // 16, _init, 0)

        def _scan(j, _):
            ids16 = ids_v[pl.ds(j * 16, 16)]
            batch = j * 16 + iota16
            _, islast = plsc.scan_count(ids16)
            loc = ids16 - nbase
            m1 = islast & (loc >= 0) & (loc < R)
            plsc.store_scatter(win_v, [jnp.clip(loc, 0, R - 1)], batch,
                               mask=m1)
            return 0
        lax.fori_loop(0, B // 16, _scan, 0)

        def _comp(i, cnt):
            w16 = win_v[pl.ds(i * 16, 16)]
            m = w16 >= 0
            n16 = nbase + i * 16 + iota16
            plsc.store_compressed(nodes_l.at[pl.ds(cnt, 16)], n16, mask=m)
            plsc.store_compressed(win_l.at[pl.ds(cnt, 16)], w16, mask=m)
            return cnt + jnp.max(plsc.all_reduce_population_count(m))
        cnt = lax.fori_loop(0, R // 16, _comp, jnp.int32(0))

        cp_ts.wait()
        cp_lu.wait()

        def _blend(i, _):
            w16 = win_v[pl.ds(i * 16, 16)]
            m = w16 >= 0
            tsv = plsc.load_gather(ts_v, [jnp.maximum(w16, 0)])
            cur = luv[pl.ds(i * 16, 16)]
            luv[pl.ds(i * 16, 16)] = jnp.where(m, tsv, cur)
            return 0
        lax.fori_loop(0, R // 16, _blend, 0)
        pltpu.sync_copy(luv, out_lu.at[pl.ds(nbase, R)])

        zero16 = jnp.zeros((16,), jnp.int32)
        padn = nodes_l[pl.ds(0, 16)].at[zero16].get(mode="promise_in_bounds")
        padw = win_l[pl.ds(0, 16)].at[zero16].get(mode="promise_in_bounds")
        npad = ((cnt + CH - 1) // CH) * CH

        def _pad(j, _):
            keep = (j * 16 + iota16) < cnt
            nodes_l[pl.ds(j * 16, 16)] = jnp.where(
                keep, nodes_l[pl.ds(j * 16, 16)], padn)
            win_l[pl.ds(j * 16, 16)] = jnp.where(
                keep, win_l[pl.ds(j * 16, 16)], padw)
            return 0
        lax.fori_loop(cnt // 16, npad // 16, _pad, 0)

        cnt_v[pl.ds(0, 16)] = jnp.full((16,), 1, jnp.int32) * cnt
        pltpu.sync_copy(cnt_v, out_counts.at[pl.ds(wid * 16, 16)])
        pltpu.sync_copy(nodes_l, out_nodes.at[pl.ds(wid * NLIST, NLIST)])
        pltpu.sync_copy(win_l, out_wins.at[pl.ds(wid * NLIST, NLIST)])


@functools.partial(
    pl.kernel,
    out_type=(),
    mesh=_mesh,
    scratch_types=[
        pltpu.VMEM((NLIST,), jnp.int32),
        pltpu.VMEM((NLIST,), jnp.int32),
        pltpu.VMEM((16,), jnp.int32),
        pltpu.VMEM((CH,), jnp.int32),
        pltpu.VMEM((CH,), jnp.int32),
        pltpu.VMEM((CH, D), jnp.float32),
        pltpu.SemaphoreType.DMA,
        pltpu.SemaphoreType.DMA,
        pltpu.SemaphoreType.DMA,
        pltpu.SemaphoreType.DMA,
        pltpu.SemaphoreType.DMA,
    ],
    compiler_params=pltpu.CompilerParams(needs_layout_passes=False),
)
def _sc_scatter(upd, nodes, wins, counts, out_mem,
                nodes_l, win_l, cnt_v, wchunk, nchunk, rowbuf,
                sem_n, sem_w, sem_c, sem_g, sem_s):
    wid = lax.axis_index("s") * NC + lax.axis_index("c")

    @pl.when(wid < NA)
    def _active():
        cp_n = pltpu.async_copy(nodes.at[pl.ds(wid * NLIST, NLIST)],
                                nodes_l, sem_n)
        cp_w = pltpu.async_copy(wins.at[pl.ds(wid * NLIST, NLIST)],
                                win_l, sem_w)
        cp_c = pltpu.async_copy(counts.at[pl.ds(wid * 16, 16)], cnt_v,
                                sem_c)
        cp_n.wait()
        cp_w.wait()
        cp_c.wait()
        cnt = jnp.max(cnt_v[pl.ds(0, 16)])
        npad = ((cnt + CH - 1) // CH) * CH

        def _chunk(k, _):
            off = k * CH
            for t in range(CH // 16):
                wchunk[pl.ds(t * 16, 16)] = win_l[pl.ds(off + t * 16, 16)]
                nchunk[pl.ds(t * 16, 16)] = nodes_l[pl.ds(off + t * 16, 16)]
            pltpu.async_copy(upd.at[wchunk], rowbuf, sem_g).wait()
            pltpu.async_copy(rowbuf, out_mem.at[nchunk], sem_s).wait()
            return 0

        @pl.when(cnt > 0)
        def _scatter_phase():
            lax.fori_loop(0, npad // CH, _chunk, 0)


def _gru_body(x_ref, h_ref, wih_ref, whh_ref, bih_ref, bhh_ref, out_ref):
    x = x_ref[...]
    h = h_ref[...]
    gi = jnp.dot(x, wih_ref[...], preferred_element_type=jnp.float32)
    gh = jnp.dot(h, whh_ref[...], preferred_element_type=jnp.float32)
    gi = gi + bih_ref[...]
    gh = gh + bhh_ref[...]
    r = jax.nn.sigmoid(gi[:, :D] + gh[:, :D])
    z = jax.nn.sigmoid(gi[:, D:2 * D] + gh[:, D:2 * D])
    n = jnp.tanh(gi[:, 2 * D:] + r * gh[:, 2 * D:])
    out_ref[...] = (1.0 - z) * n + z * h


_RB = 2048


def _gru(messages, h, W_ihT, W_hhT, b_ih2, b_hh2):
    return pl.pallas_call(
        _gru_body,
        grid=(B // _RB,),
        in_specs=[
            pl.BlockSpec((_RB, IN_DIM), lambda i: (i, 0)),
            pl.BlockSpec((_RB, D), lambda i: (i, 0)),
            pl.BlockSpec((IN_DIM, 3 * D), lambda i: (0, 0)),
            pl.BlockSpec((D, 3 * D), lambda i: (0, 0)),
            pl.BlockSpec((1, 3 * D), lambda i: (0, 0)),
            pl.BlockSpec((1, 3 * D), lambda i: (0, 0)),
        ],
        out_specs=pl.BlockSpec((_RB, D), lambda i: (i, 0)),
        out_shape=jax.ShapeDtypeStruct((B, D), jnp.float32),
    )(messages, h, W_ihT, W_hhT, b_ih2, b_hh2)


def kernel(node_ids, messages, timestamps, memory, last_update, W_ih, W_hh, b_ih, b_hh):
    ids = node_ids.astype(jnp.int32)
    h = _sc_gather(memory, ids)
    plan_nodes, plan_wins, plan_counts, new_last_update = _sc_plan(
        ids, timestamps, last_update)
    updated = _gru(messages, h, W_ih.T, W_hh.T,
                   b_ih.reshape(1, -1), b_hh.reshape(1, -1))
    out_mem_ref = jax.new_ref(memory)
    _sc_scatter(updated, plan_nodes, plan_wins, plan_counts, out_mem_ref)
    new_memory = out_mem_ref[...]
    return new_memory, new_last_update
